# Initial kernel scaffold; baseline (speedup 1.0000x reference)
import functools

import jax
import jax.numpy as jnp
from jax import lax
from jax.experimental import pallas as pl
from jax.experimental.pallas import tpu as pltpu
from jax.experimental.pallas import tpu_sc as plsc

N = 10000
E = 320000
F = 128
H = 256
C = 64

NC = 2
NS = 16
L = 16

NHIST = 10240
NAGG = 10016
ROWS_PER_TILE = NAGG // NS

CHUNK = 128
CPT = 80
EPC = NS * CPT * CHUNK
EPAD = NC * EPC

_VMESH = plsc.VectorSubcoreMesh(core_axis_name="c", subcore_axis_name="s")


def _degree_body(edges_hbm, out_hbm, idx_v, hist_v, sem):
    c = lax.axis_index("c")
    s = lax.axis_index("s")
    epw = E // NS

    @pl.loop(0, NHIST, step=L)
    def _(i):
        hist_v[pl.ds(i, L)] = jnp.zeros((L,), jnp.float32)

    pltpu.async_copy(edges_hbm.at[c, pl.ds(s * epw, epw)], idx_v, sem).wait()
    ones = jnp.full((L,), 1.0, jnp.float32)

    @pl.loop(0, epw, step=L)
    def _(i):
        iv = idx_v[pl.ds(i, L)]
        plsc.addupdate_scatter(hist_v, [iv], ones)

    pltpu.async_copy(hist_v, out_hbm.at[c, s], sem).wait()


@jax.jit
def _degrees(edge_index):
    k = pl.kernel(
        _degree_body,
        out_type=jax.ShapeDtypeStruct((NC, NS, NHIST), jnp.float32),
        mesh=_VMESH,
        scratch_types=[
            pltpu.VMEM((E // NS,), jnp.int32),
            pltpu.VMEM((NHIST,), jnp.float32),
            pltpu.SemaphoreType.DMA,
        ],
    )
    return k(edge_index)


def _norms_body(degp_ref, norms_ref):
    deg = jnp.sum(degp_ref[...], axis=1)
    lane = lax.broadcasted_iota(jnp.int32, (NC, NHIST), 1)
    norm = lax.rsqrt(jnp.maximum(deg, 1.0))
    norms_ref[...] = jnp.where(lane < N, norm, 0.0)


@jax.jit
def _norms(degp):
    return pl.pallas_call(
        _norms_body,
        out_shape=jax.ShapeDtypeStruct((NC, NHIST), jnp.float32),
    )(degp)


def _scale_body(x_ref, onorm_ref, xs_ref):
    xs_ref[...] = x_ref[...] * onorm_ref[...]


@jax.jit
def _scale(in_feat, onorm_col):
    return pl.pallas_call(
        _scale_body,
        out_shape=jax.ShapeDtypeStruct((N, F), jnp.float32),
    )(in_feat, onorm_col)


def _agg_body(xs_hbm, srcc, dstc, inorm_hbm, zeros_hbm, agg_hbm, wps_hbm,
              agg_sp, src_v, dst_v, inorm_v, whist_v, buf0, buf1, sem_i,
              sem_g):
    c = lax.axis_index("c")
    s = lax.axis_index("s")
    base = s * ROWS_PER_TILE

    pltpu.async_copy(srcc.at[c, s], src_v, sem_i).wait()
    pltpu.async_copy(dstc.at[c, s], dst_v, sem_i).wait()
    pltpu.async_copy(inorm_hbm, inorm_v, sem_i).wait()

    pltpu.async_copy(zeros_hbm.at[pl.ds(base, ROWS_PER_TILE)],
                     agg_sp.at[pl.ds(base, ROWS_PER_TILE)], sem_i).wait()

    @pl.loop(0, NHIST, step=L)
    def _(i):
        whist_v[pl.ds(i, L)] = jnp.zeros((L,), jnp.float32)

    plsc.subcore_barrier()

    @pl.loop(0, CPT)
    def _(ch):
        @pl.loop(0, CHUNK, step=L)
        def _(k):
            dv = dst_v[ch, pl.ds(k, L)]
            sv = src_v[ch, pl.ds(k, L)]
            vals = plsc.load_gather(inorm_v, [dv])
            plsc.addupdate_scatter(whist_v, [sv], vals)

    pltpu.async_copy(xs_hbm.at[src_v.at[0]], buf0, sem_g)

    @pl.loop(0, CPT, step=2)
    def _(ch):
        pltpu.make_async_copy(xs_hbm.at[src_v.at[ch]], buf0, sem_g).wait()
        pltpu.async_copy(xs_hbm.at[src_v.at[ch + 1]], buf1, sem_g)
        pltpu.sync_copy(buf0, agg_sp.at[dst_v.at[ch]], add=True)
        pltpu.make_async_copy(xs_hbm.at[src_v.at[ch + 1]], buf1, sem_g).wait()

        @pl.when(ch + 2 < CPT)
        def _():
            pltpu.async_copy(xs_hbm.at[src_v.at[ch + 2]], buf0, sem_g)

        pltpu.sync_copy(buf1, agg_sp.at[dst_v.at[ch + 1]], add=True)

    plsc.subcore_barrier()

    pltpu.async_copy(agg_sp.at[pl.ds(base, ROWS_PER_TILE)],
                     agg_hbm.at[c, pl.ds(base, ROWS_PER_TILE)], sem_i).wait()
    pltpu.async_copy(whist_v, wps_hbm.at[c, s], sem_i).wait()


@jax.jit
def _aggregate(xs, srcc, dstc, inorm_flat, zeros_rows):
    k = pl.kernel(
        _agg_body,
        out_type=(
            jax.ShapeDtypeStruct((NC, NAGG, F), jnp.float32),
            jax.ShapeDtypeStruct((NC, NS, NHIST), jnp.float32),
        ),
        mesh=_VMESH,
        scratch_types=[
            pltpu.VMEM_SHARED((NAGG, F), jnp.float32),
            pltpu.VMEM((CPT, CHUNK), jnp.int32),
            pltpu.VMEM((CPT, CHUNK), jnp.int32),
            pltpu.VMEM((NHIST,), jnp.float32),
            pltpu.VMEM((NHIST,), jnp.float32),
            pltpu.VMEM((CHUNK, F), jnp.float32),
            pltpu.VMEM((CHUNK, F), jnp.float32),
            pltpu.SemaphoreType.DMA,
            pltpu.SemaphoreType.DMA,
        ],
    )
    return k(xs, srcc, dstc, inorm_flat, zeros_rows)


def _final_body(agg_ref, inorm_ref, onorm_ref, wps_ref, w1_ref, b1_ref,
                w2_ref, b2_ref, out_ref):
    agg = agg_ref[0, :N, :] + agg_ref[1, :N, :]
    a = agg * inorm_ref[...]
    z = jnp.dot(a, w1_ref[...], preferred_element_type=jnp.float32,
                precision=lax.Precision.HIGHEST) + b1_ref[...]
    h = jnp.maximum(z, 0.0)
    wpre = jnp.sum(wps_ref[...], axis=0, keepdims=True)[:, :N]
    w_row = wpre * onorm_ref[...]
    v = jnp.dot(w_row, h, preferred_element_type=jnp.float32,
                precision=lax.Precision.HIGHEST) * (1.0 / N)
    out_ref[...] = jnp.dot(v, w2_ref[...], preferred_element_type=jnp.float32,
                           precision=lax.Precision.HIGHEST) + b2_ref[...]


@jax.jit
def _final(agg, inorm_col, onorm_row, wps2, W1, b1, W2, b2):
    return pl.pallas_call(
        _final_body,
        out_shape=jax.ShapeDtypeStruct((1, C), jnp.float32),
    )(agg, inorm_col, onorm_row, wps2, W1, b1, W2, b2)


def kernel(in_feat, edge_index, W1, b1, W2, b2):
    src = edge_index[0]
    dst = edge_index[1]

    degp = _degrees(edge_index)
    norms = _norms(degp)

    onorm_col = norms[0].reshape(NHIST, 1)[:N]
    inorm_col = norms[1].reshape(NHIST, 1)[:N]
    onorm_row = norms[0:1, :N]
    inorm_flat = norms[1]

    xs = _scale(in_feat, onorm_col)

    pad_src = jnp.zeros((EPAD - E,), jnp.int32)
    pad_dst = jnp.full((EPAD - E,), N, jnp.int32)
    srcc = jnp.concatenate([src, pad_src]).reshape(NC, NS, CPT, CHUNK)
    dstc = jnp.concatenate([dst, pad_dst]).reshape(NC, NS, CPT, CHUNK)
    zeros_rows = jnp.zeros((NAGG, F), jnp.float32)

    agg, wps = _aggregate(xs, srcc, dstc, inorm_flat, zeros_rows)

    wps2 = wps.reshape(NC * NS, NHIST)
    out = _final(agg, inorm_col, onorm_row, wps2, W1,
                 b1.reshape(1, H), W2, b2.reshape(1, C))
    return out


# speedup vs baseline: 6.7755x; 6.7755x over previous
import dataclasses
import functools

import jax
import jax.numpy as jnp
from jax import lax
from jax.experimental import pallas as pl
from jax.experimental.pallas import tpu as pltpu
from jax.experimental.pallas import tpu_sc as plsc

N = 10000
E = 320000
F = 128
H = 256
C = 64

NC = 2
NS = 16
L = 16

NHIST = 10240
NAGG = 10112
ROWS_PER_TILE = NAGG // NS

CHUNK = 128
BCH = 8
NBLK = 10
CPT = NBLK * BCH
EPC = NS * CPT * CHUNK
EPAD = NC * EPC

_VMESH = plsc.VectorSubcoreMesh(core_axis_name="c", subcore_axis_name="s")

_SC_PARAMS = pltpu.CompilerParams()
if "needs_layout_passes" in pltpu.CompilerParams.__dataclass_fields__:
    _SC_PARAMS = dataclasses.replace(_SC_PARAMS, needs_layout_passes=False)


def _degree_body(edges_hbm, out_hbm, idx_v, hist_v, sem):
    c = lax.axis_index("c")
    s = lax.axis_index("s")
    epw = E // NS

    @pl.loop(0, NHIST, step=L)
    def _(i):
        hist_v[pl.ds(i, L)] = jnp.zeros((L,), jnp.float32)

    pltpu.async_copy(edges_hbm.at[pl.ds(c * E + s * epw, epw)], idx_v,
                     sem).wait()
    ones = jnp.full((L,), 1.0, jnp.float32)

    @pl.loop(0, epw, step=L)
    def _(i):
        iv = idx_v[pl.ds(i, L)]
        plsc.addupdate_scatter(hist_v, [iv], ones)

    pltpu.async_copy(hist_v, out_hbm.at[c, s], sem).wait()


@jax.jit
def _degrees(edge_index):
    k = pl.kernel(
        _degree_body,
        out_type=jax.ShapeDtypeStruct((NC, NS, NHIST), jnp.float32),
        mesh=_VMESH,
        compiler_params=_SC_PARAMS,
        scratch_types=[
            pltpu.VMEM((E // NS,), jnp.int32),
            pltpu.VMEM((NHIST,), jnp.float32),
            pltpu.SemaphoreType.DMA,
        ],
    )
    return k(edge_index.reshape(2 * E))


def _norms_body(degp_ref, norms_ref):
    deg = jnp.sum(degp_ref[...], axis=1)
    lane = lax.broadcasted_iota(jnp.int32, (NC, NHIST), 1)
    norm = lax.rsqrt(jnp.maximum(deg, 1.0))
    norms_ref[...] = jnp.where(lane < N, norm, 0.0)


@jax.jit
def _norms(degp):
    return pl.pallas_call(
        _norms_body,
        out_shape=jax.ShapeDtypeStruct((NC, NHIST), jnp.float32),
    )(degp)


def _scale_body(x_ref, onorm_ref, xs_ref):
    xs_ref[...] = x_ref[...] * onorm_ref[...]


@jax.jit
def _scale(in_feat, onorm_col):
    return pl.pallas_call(
        _scale_body,
        out_shape=jax.ShapeDtypeStruct((N, F), jnp.float32),
    )(in_feat, onorm_col)


def _w_body(edges_hbm, inorm_hbm, wps_hbm, src_v, dst_v, inorm_v, whist_v,
            sem):
    c = lax.axis_index("c")
    s = lax.axis_index("s")
    epw = E // (NC * NS)
    w = c * NS + s

    @pl.loop(0, NHIST, step=L)
    def _(i):
        whist_v[pl.ds(i, L)] = jnp.zeros((L,), jnp.float32)

    pltpu.async_copy(edges_hbm.at[pl.ds(w * epw, epw)], src_v, sem).wait()
    pltpu.async_copy(edges_hbm.at[pl.ds(E + w * epw, epw)], dst_v, sem).wait()
    pltpu.async_copy(inorm_hbm, inorm_v, sem).wait()

    @pl.loop(0, epw, step=L)
    def _(i):
        dv = dst_v[pl.ds(i, L)]
        sv = src_v[pl.ds(i, L)]
        vals = plsc.load_gather(inorm_v, [dv])
        plsc.addupdate_scatter(whist_v, [sv], vals)

    pltpu.async_copy(whist_v, wps_hbm.at[c, s], sem).wait()


@jax.jit
def _w_partials(edges_flat, inorm_flat):
    k = pl.kernel(
        _w_body,
        out_type=jax.ShapeDtypeStruct((NC, NS, NHIST), jnp.float32),
        mesh=_VMESH,
        compiler_params=_SC_PARAMS,
        scratch_types=[
            pltpu.VMEM((E // (NC * NS),), jnp.int32),
            pltpu.VMEM((E // (NC * NS),), jnp.int32),
            pltpu.VMEM((NHIST,), jnp.float32),
            pltpu.VMEM((NHIST,), jnp.float32),
            pltpu.SemaphoreType.DMA,
        ],
    )
    return k(edges_flat, inorm_flat)


def _agg_body(xs_hbm, srcc, dstc, zeros_hbm, agg_hbm, agg_sp,
              sblk0, sblk1, dblk0, dblk1, buf0, buf1, sem_i, sem_x, sem_g):
    c = lax.axis_index("c")
    s = lax.axis_index("s")
    base = s * ROWS_PER_TILE

    pltpu.async_copy(zeros_hbm.at[pl.ds(base, ROWS_PER_TILE)],
                     agg_sp.at[pl.ds(base, ROWS_PER_TILE)], sem_i).wait()

    pltpu.async_copy(srcc.at[c, s, 0], sblk0, sem_i).wait()
    pltpu.async_copy(dstc.at[c, s, 0], dblk0, sem_i).wait()
    plsc.subcore_barrier()
    pltpu.async_copy(xs_hbm.at[sblk0.at[0]], buf0, sem_g)

    def _block(b, sblk, dblk, sblk_nxt, dblk_nxt):
        @pl.when(b + 1 < NBLK)
        def _():
            pltpu.async_copy(srcc.at[c, s, b + 1], sblk_nxt, sem_x)
            pltpu.async_copy(dstc.at[c, s, b + 1], dblk_nxt, sem_x)

        for k in range(BCH):
            bufa, bufb = (buf0, buf1) if k % 2 == 0 else (buf1, buf0)
            pltpu.make_async_copy(xs_hbm.at[sblk.at[k]], bufa, sem_g).wait()
            if k + 1 < BCH:
                pltpu.async_copy(xs_hbm.at[sblk.at[k + 1]], bufb, sem_g)
            else:
                @pl.when(b + 1 < NBLK)
                def _():
                    pltpu.make_async_copy(srcc.at[c, s, b + 1], sblk_nxt,
                                          sem_x).wait()
                    pltpu.make_async_copy(dstc.at[c, s, b + 1], dblk_nxt,
                                          sem_x).wait()
                    pltpu.async_copy(xs_hbm.at[sblk_nxt.at[0]], bufb, sem_g)
            pltpu.sync_copy(bufa, agg_sp.at[dblk.at[k]], add=True)

    @pl.loop(0, NBLK, step=2)
    def _(b):
        _block(b, sblk0, dblk0, sblk1, dblk1)
        _block(b + 1, sblk1, dblk1, sblk0, dblk0)

    plsc.subcore_barrier()

    pltpu.async_copy(agg_sp.at[pl.ds(base, ROWS_PER_TILE)],
                     agg_hbm.at[c, pl.ds(base, ROWS_PER_TILE)], sem_i).wait()


@jax.jit
def _aggregate(xs, srcc, dstc, zeros_rows):
    k = pl.kernel(
        _agg_body,
        out_type=jax.ShapeDtypeStruct((NC, NAGG, F), jnp.float32),
        mesh=_VMESH,
        compiler_params=_SC_PARAMS,
        scratch_types=[
            pltpu.VMEM_SHARED((NAGG, F), jnp.float32),
            pltpu.VMEM((BCH, CHUNK), jnp.int32),
            pltpu.VMEM((BCH, CHUNK), jnp.int32),
            pltpu.VMEM((BCH, CHUNK), jnp.int32),
            pltpu.VMEM((BCH, CHUNK), jnp.int32),
            pltpu.VMEM((CHUNK, F), jnp.float32),
            pltpu.VMEM((CHUNK, F), jnp.float32),
            pltpu.SemaphoreType.DMA,
            pltpu.SemaphoreType.DMA,
            pltpu.SemaphoreType.DMA,
        ],
    )
    return k(xs, srcc, dstc, zeros_rows)


def _final_body(agg_ref, inorm_ref, onorm_ref, wps_ref, w1_ref, b1_ref,
                w2_ref, b2_ref, out_ref):
    agg = agg_ref[0, :N, :] + agg_ref[1, :N, :]
    a = agg * inorm_ref[...]
    z = jnp.dot(a, w1_ref[...], preferred_element_type=jnp.float32,
                precision=lax.Precision.HIGHEST) + b1_ref[...]
    h = jnp.maximum(z, 0.0)
    wpre = jnp.sum(wps_ref[...], axis=0, keepdims=True)[:, :N]
    w_row = wpre * onorm_ref[...]
    v = jnp.dot(w_row, h, preferred_element_type=jnp.float32,
                precision=lax.Precision.HIGHEST) * (1.0 / N)
    out_ref[...] = jnp.dot(v, w2_ref[...], preferred_element_type=jnp.float32,
                           precision=lax.Precision.HIGHEST) + b2_ref[...]


@jax.jit
def _final(agg, inorm_col, onorm_row, wps2, W1, b1, W2, b2):
    return pl.pallas_call(
        _final_body,
        out_shape=jax.ShapeDtypeStruct((1, C), jnp.float32),
    )(agg, inorm_col, onorm_row, wps2, W1, b1, W2, b2)


def kernel(in_feat, edge_index, W1, b1, W2, b2):
    src = edge_index[0]
    dst = edge_index[1]

    degp = _degrees(edge_index)
    norms = _norms(degp)

    onorm_col = norms[0].reshape(NHIST, 1)[:N]
    inorm_col = norms[1].reshape(NHIST, 1)[:N]
    onorm_row = norms[0:1, :N]
    inorm_flat = norms[1]

    xs = _scale(in_feat, onorm_col)
    wps = _w_partials(edge_index.reshape(2 * E), inorm_flat)

    pad_src = jnp.zeros((EPAD - E,), jnp.int32)
    pad_dst = jnp.full((EPAD - E,), N, jnp.int32)
    srcc = jnp.concatenate([src, pad_src]).reshape(NC, NS, NBLK, BCH, CHUNK)
    dstc = jnp.concatenate([dst, pad_dst]).reshape(NC, NS, NBLK, BCH, CHUNK)
    zeros_rows = jnp.zeros((NAGG, F), jnp.float32)

    agg = _aggregate(xs, srcc, dstc, zeros_rows)

    wps2 = wps.reshape(NC * NS, NHIST)
    out = _final(agg, inorm_col, onorm_row, wps2, W1,
                 b1.reshape(1, H), W2, b2.reshape(1, C))
    return out


# speedup vs baseline: 6.7814x; 1.0009x over previous
import dataclasses
import functools

import jax
import jax.numpy as jnp
from jax import lax
from jax.experimental import pallas as pl
from jax.experimental.pallas import tpu as pltpu
from jax.experimental.pallas import tpu_sc as plsc

N = 10000
E = 320000
F = 128
H = 256
C = 64

NC = 2
NS = 16
L = 16

NHIST = 10240
NAGG = 10112
ROWS_PER_TILE = NAGG // NS

CHUNK = 128
BCH = 8
NBLK = 10
CPT = NBLK * BCH
EPC = NS * CPT * CHUNK
EPAD = NC * EPC

_VMESH = plsc.VectorSubcoreMesh(core_axis_name="c", subcore_axis_name="s")

_SC_PARAMS = pltpu.CompilerParams()
if "needs_layout_passes" in pltpu.CompilerParams.__dataclass_fields__:
    _SC_PARAMS = dataclasses.replace(_SC_PARAMS, needs_layout_passes=False)


def _degree_body(edges_hbm, out_hbm, idx_v, hist_v, sem):
    c = lax.axis_index("c")
    s = lax.axis_index("s")
    epw = E // NS

    @pl.loop(0, NHIST, step=L)
    def _(i):
        hist_v[pl.ds(i, L)] = jnp.zeros((L,), jnp.float32)

    pltpu.async_copy(edges_hbm.at[pl.ds(c * E + s * epw, epw)], idx_v,
                     sem).wait()
    ones = jnp.full((L,), 1.0, jnp.float32)

    @pl.loop(0, epw, step=L)
    def _(i):
        iv = idx_v[pl.ds(i, L)]
        plsc.addupdate_scatter(hist_v, [iv], ones)

    pltpu.async_copy(hist_v, out_hbm.at[c, s], sem).wait()


@jax.jit
def _degrees(edge_index):
    k = pl.kernel(
        _degree_body,
        out_type=jax.ShapeDtypeStruct((NC, NS, NHIST), jnp.float32),
        mesh=_VMESH,
        compiler_params=_SC_PARAMS,
        scratch_types=[
            pltpu.VMEM((E // NS,), jnp.int32),
            pltpu.VMEM((NHIST,), jnp.float32),
            pltpu.SemaphoreType.DMA,
        ],
    )
    return k(edge_index.reshape(2 * E))


def _norms_body(degp_ref, norms_ref):
    deg = jnp.sum(degp_ref[...], axis=1)
    lane = lax.broadcasted_iota(jnp.int32, (NC, NHIST), 1)
    norm = lax.rsqrt(jnp.maximum(deg, 1.0))
    norms_ref[...] = jnp.where(lane < N, norm, 0.0)


@jax.jit
def _norms(degp):
    return pl.pallas_call(
        _norms_body,
        out_shape=jax.ShapeDtypeStruct((NC, NHIST), jnp.float32),
    )(degp)


def _scale_body(x_ref, onorm_ref, xs_ref):
    xs_ref[...] = x_ref[...] * onorm_ref[...]


@jax.jit
def _scale(in_feat, onorm_col):
    return pl.pallas_call(
        _scale_body,
        out_shape=jax.ShapeDtypeStruct((N, F), jnp.float32),
    )(in_feat, onorm_col)


def _w_body(edges_hbm, inorm_hbm, wps_hbm, src_v, dst_v, inorm_v, whist_v,
            sem):
    c = lax.axis_index("c")
    s = lax.axis_index("s")
    epw = E // (NC * NS)
    w = c * NS + s

    @pl.loop(0, NHIST, step=L)
    def _(i):
        whist_v[pl.ds(i, L)] = jnp.zeros((L,), jnp.float32)

    pltpu.async_copy(edges_hbm.at[pl.ds(w * epw, epw)], src_v, sem).wait()
    pltpu.async_copy(edges_hbm.at[pl.ds(E + w * epw, epw)], dst_v, sem).wait()
    pltpu.async_copy(inorm_hbm, inorm_v, sem).wait()

    @pl.loop(0, epw, step=L)
    def _(i):
        dv = dst_v[pl.ds(i, L)]
        sv = src_v[pl.ds(i, L)]
        vals = plsc.load_gather(inorm_v, [dv])
        plsc.addupdate_scatter(whist_v, [sv], vals)

    pltpu.async_copy(whist_v, wps_hbm.at[c, s], sem).wait()


@jax.jit
def _w_partials(edges_flat, inorm_flat):
    k = pl.kernel(
        _w_body,
        out_type=jax.ShapeDtypeStruct((NC, NS, NHIST), jnp.float32),
        mesh=_VMESH,
        compiler_params=_SC_PARAMS,
        scratch_types=[
            pltpu.VMEM((E // (NC * NS),), jnp.int32),
            pltpu.VMEM((E // (NC * NS),), jnp.int32),
            pltpu.VMEM((NHIST,), jnp.float32),
            pltpu.VMEM((NHIST,), jnp.float32),
            pltpu.SemaphoreType.DMA,
        ],
    )
    return k(edges_flat, inorm_flat)


def _agg_body(xs_hbm, srcc, dstc, zeros_hbm, agg_hbm, agg_sp,
              sblk0, sblk1, dblk0, dblk1, buf0, buf1, sem_i, sem_x, sem_g,
              sem_s0, sem_s1):
    c = lax.axis_index("c")
    s = lax.axis_index("s")
    base = s * ROWS_PER_TILE

    pltpu.async_copy(zeros_hbm.at[pl.ds(base, ROWS_PER_TILE)],
                     agg_sp.at[pl.ds(base, ROWS_PER_TILE)], sem_i).wait()

    pltpu.async_copy(srcc.at[c, s, 0], sblk0, sem_i).wait()
    pltpu.async_copy(dstc.at[c, s, 0], dblk0, sem_i).wait()
    plsc.subcore_barrier()
    pltpu.async_copy(xs_hbm.at[sblk0.at[0]], buf0, sem_g)

    def _block(b, sblk, dblk, sblk_nxt, dblk_nxt):
        for k in range(BCH):
            if k % 2 == 0:
                bufa, bufb, sema, semb = buf0, buf1, sem_s0, sem_s1
            else:
                bufa, bufb, sema, semb = buf1, buf0, sem_s1, sem_s0
            pltpu.make_async_copy(xs_hbm.at[sblk.at[k]], bufa, sem_g).wait()
            pltpu.async_copy(bufa, agg_sp.at[dblk.at[k]], sema, add=True)
            if k == 0:
                @pl.when(b > 0)
                def _():
                    pltpu.make_async_copy(bufb, agg_sp.at[dblk.at[k]],
                                          semb).wait()
            else:
                pltpu.make_async_copy(bufb, agg_sp.at[dblk.at[k]],
                                      semb).wait()
            if k == 1:
                @pl.when(b + 1 < NBLK)
                def _():
                    pltpu.async_copy(srcc.at[c, s, b + 1], sblk_nxt, sem_x)
                    pltpu.async_copy(dstc.at[c, s, b + 1], dblk_nxt, sem_x)
            if k + 1 < BCH:
                pltpu.async_copy(xs_hbm.at[sblk.at[k + 1]], bufb, sem_g)
            else:
                @pl.when(b + 1 < NBLK)
                def _():
                    pltpu.make_async_copy(srcc.at[c, s, b + 1], sblk_nxt,
                                          sem_x).wait()
                    pltpu.make_async_copy(dstc.at[c, s, b + 1], dblk_nxt,
                                          sem_x).wait()
                    pltpu.async_copy(xs_hbm.at[sblk_nxt.at[0]], bufb, sem_g)

    @pl.loop(0, NBLK, step=2)
    def _(b):
        _block(b, sblk0, dblk0, sblk1, dblk1)
        _block(b + 1, sblk1, dblk1, sblk0, dblk0)

    pltpu.make_async_copy(buf1, agg_sp.at[dblk1.at[BCH - 1]], sem_s1).wait()
    plsc.subcore_barrier()

    pltpu.async_copy(agg_sp.at[pl.ds(base, ROWS_PER_TILE)],
                     agg_hbm.at[c, pl.ds(base, ROWS_PER_TILE)], sem_i).wait()


@jax.jit
def _aggregate(xs, srcc, dstc, zeros_rows):
    k = pl.kernel(
        _agg_body,
        out_type=jax.ShapeDtypeStruct((NC, NAGG, F), jnp.float32),
        mesh=_VMESH,
        compiler_params=_SC_PARAMS,
        scratch_types=[
            pltpu.VMEM_SHARED((NAGG, F), jnp.float32),
            pltpu.VMEM((BCH, CHUNK), jnp.int32),
            pltpu.VMEM((BCH, CHUNK), jnp.int32),
            pltpu.VMEM((BCH, CHUNK), jnp.int32),
            pltpu.VMEM((BCH, CHUNK), jnp.int32),
            pltpu.VMEM((CHUNK, F), jnp.float32),
            pltpu.VMEM((CHUNK, F), jnp.float32),
            pltpu.SemaphoreType.DMA,
            pltpu.SemaphoreType.DMA,
            pltpu.SemaphoreType.DMA,
            pltpu.SemaphoreType.DMA,
            pltpu.SemaphoreType.DMA,
        ],
    )
    return k(xs, srcc, dstc, zeros_rows)


def _final_body(agg_ref, inorm_ref, onorm_ref, wps_ref, w1_ref, b1_ref,
                w2_ref, b2_ref, out_ref):
    agg = agg_ref[0, :N, :] + agg_ref[1, :N, :]
    a = agg * inorm_ref[...]
    z = jnp.dot(a, w1_ref[...], preferred_element_type=jnp.float32,
                precision=lax.Precision.HIGHEST) + b1_ref[...]
    h = jnp.maximum(z, 0.0)
    wpre = jnp.sum(wps_ref[...], axis=0, keepdims=True)[:, :N]
    w_row = wpre * onorm_ref[...]
    v = jnp.dot(w_row, h, preferred_element_type=jnp.float32,
                precision=lax.Precision.HIGHEST) * (1.0 / N)
    out_ref[...] = jnp.dot(v, w2_ref[...], preferred_element_type=jnp.float32,
                           precision=lax.Precision.HIGHEST) + b2_ref[...]


@jax.jit
def _final(agg, inorm_col, onorm_row, wps2, W1, b1, W2, b2):
    return pl.pallas_call(
        _final_body,
        out_shape=jax.ShapeDtypeStruct((1, C), jnp.float32),
    )(agg, inorm_col, onorm_row, wps2, W1, b1, W2, b2)


def kernel(in_feat, edge_index, W1, b1, W2, b2):
    src = edge_index[0]
    dst = edge_index[1]

    degp = _degrees(edge_index)
    norms = _norms(degp)

    onorm_col = norms[0].reshape(NHIST, 1)[:N]
    inorm_col = norms[1].reshape(NHIST, 1)[:N]
    onorm_row = norms[0:1, :N]
    inorm_flat = norms[1]

    xs = _scale(in_feat, onorm_col)
    wps = _w_partials(edge_index.reshape(2 * E), inorm_flat)

    pad_src = jnp.zeros((EPAD - E,), jnp.int32)
    pad_dst = jnp.full((EPAD - E,), N, jnp.int32)
    srcc = jnp.concatenate([src, pad_src]).reshape(NC, NS, NBLK, BCH, CHUNK)
    dstc = jnp.concatenate([dst, pad_dst]).reshape(NC, NS, NBLK, BCH, CHUNK)
    zeros_rows = jnp.zeros((NAGG, F), jnp.float32)

    agg = _aggregate(xs, srcc, dstc, zeros_rows)

    wps2 = wps.reshape(NC * NS, NHIST)
    out = _final(agg, inorm_col, onorm_row, wps2, W1,
                 b1.reshape(1, H), W2, b2.reshape(1, C))
    return out


# speedup vs baseline: 10.3055x; 1.5197x over previous
import dataclasses
import functools

import jax
import jax.numpy as jnp
from jax import lax
from jax.experimental import pallas as pl
from jax.experimental.pallas import tpu as pltpu
from jax.experimental.pallas import tpu_sc as plsc

N = 10000
E = 320000
F = 128
H = 256
C = 64

NC = 2
NS = 16
L = 16

NHIST = 10240
NPADX = 10240
NB = 8
RNG = NPADX // NB
PHASES = NB // NC
EPW = E // (NC * NS)
CHUNK = 128
PAIR = 2 * CHUNK
CAPW = 10496
ZROW = N

_VMESH = plsc.VectorSubcoreMesh(core_axis_name="c", subcore_axis_name="s")

_SC_PARAMS = pltpu.CompilerParams()
if "needs_layout_passes" in pltpu.CompilerParams.__dataclass_fields__:
    _SC_PARAMS = dataclasses.replace(_SC_PARAMS, needs_layout_passes=False)


def _degree_body(edges_hbm, out_hbm, idx_v, hist_v, sem):
    c = lax.axis_index("c")
    s = lax.axis_index("s")
    epw = E // NS

    @pl.loop(0, NHIST, step=L)
    def _(i):
        hist_v[pl.ds(i, L)] = jnp.zeros((L,), jnp.float32)

    pltpu.async_copy(edges_hbm.at[pl.ds(c * E + s * epw, epw)], idx_v,
                     sem).wait()
    ones = jnp.full((L,), 1.0, jnp.float32)

    @pl.loop(0, epw, step=L)
    def _(i):
        iv = idx_v[pl.ds(i, L)]
        plsc.addupdate_scatter(hist_v, [iv], ones)

    pltpu.async_copy(hist_v, out_hbm.at[c, s], sem).wait()


@jax.jit
def _degrees(edge_index):
    k = pl.kernel(
        _degree_body,
        out_type=jax.ShapeDtypeStruct((NC, NS, NHIST), jnp.float32),
        mesh=_VMESH,
        compiler_params=_SC_PARAMS,
        scratch_types=[
            pltpu.VMEM((E // NS,), jnp.int32),
            pltpu.VMEM((NHIST,), jnp.float32),
            pltpu.SemaphoreType.DMA,
        ],
    )
    return k(edge_index.reshape(2 * E))


def _norms_body(degp_ref, norms_ref):
    deg = jnp.sum(degp_ref[...], axis=1)
    lane = lax.broadcasted_iota(jnp.int32, (NC, NHIST), 1)
    norm = lax.rsqrt(jnp.maximum(deg, 1.0))
    norms_ref[...] = jnp.where(lane < N, norm, 0.0)


@jax.jit
def _norms(degp):
    return pl.pallas_call(
        _norms_body,
        out_shape=jax.ShapeDtypeStruct((NC, NHIST), jnp.float32),
    )(degp)


def _scale_body(x_ref, onorm_ref, xs_ref):
    xs_ref[...] = x_ref[...] * onorm_ref[...]


@jax.jit
def _scale(in_feat, onorm_col):
    return pl.pallas_call(
        _scale_body,
        out_shape=jax.ShapeDtypeStruct((N, F), jnp.float32),
    )(in_feat, onorm_col)


def _w_body(edges_hbm, inorm_hbm, wps_hbm, src_v, dst_v, inorm_v, whist_v,
            sem):
    c = lax.axis_index("c")
    s = lax.axis_index("s")
    w = c * NS + s

    @pl.loop(0, NHIST, step=L)
    def _(i):
        whist_v[pl.ds(i, L)] = jnp.zeros((L,), jnp.float32)

    pltpu.async_copy(edges_hbm.at[pl.ds(w * EPW, EPW)], src_v, sem).wait()
    pltpu.async_copy(edges_hbm.at[pl.ds(E + w * EPW, EPW)], dst_v,
                     sem).wait()
    pltpu.async_copy(inorm_hbm, inorm_v, sem).wait()

    @pl.loop(0, EPW, step=L)
    def _(i):
        dv = dst_v[pl.ds(i, L)]
        sv = src_v[pl.ds(i, L)]
        vals = plsc.load_gather(inorm_v, [dv])
        plsc.addupdate_scatter(whist_v, [sv], vals)

    pltpu.async_copy(whist_v, wps_hbm.at[c, s], sem).wait()


@jax.jit
def _w_partials(edges_flat, inorm_flat):
    k = pl.kernel(
        _w_body,
        out_type=jax.ShapeDtypeStruct((NC, NS, NHIST), jnp.float32),
        mesh=_VMESH,
        compiler_params=_SC_PARAMS,
        scratch_types=[
            pltpu.VMEM((EPW,), jnp.int32),
            pltpu.VMEM((EPW,), jnp.int32),
            pltpu.VMEM((NHIST,), jnp.float32),
            pltpu.VMEM((NHIST,), jnp.float32),
            pltpu.SemaphoreType.DMA,
        ],
    )
    return k(edges_flat, inorm_flat)


def _bucket_body(edges_hbm, bsrc_hbm, bdst_hbm, bcnt_hbm,
                 src_v, dst_v, osrc_v, odst_v, cnts_v, cnt_sm, sem):
    c = lax.axis_index("c")
    s = lax.axis_index("s")
    w = c * NS + s

    pltpu.async_copy(edges_hbm.at[pl.ds(w * EPW, EPW)], src_v, sem).wait()
    pltpu.async_copy(edges_hbm.at[pl.ds(E + w * EPW, EPW)], dst_v,
                     sem).wait()

    for g in range(NB):
        cnt_sm[0] = 0

        @pl.loop(0, EPW, step=L)
        def _(i):
            dv = dst_v[pl.ds(i, L)]
            sv = src_v[pl.ds(i, L)]
            bv = lax.shift_right_logical(dv * 13108, 24)
            m = bv == g
            dlv = dv - g * RNG
            cnt = cnt_sm[0]
            plsc.store_compressed(osrc_v.at[pl.ds(cnt, L)], sv, mask=m)
            plsc.store_compressed(odst_v.at[pl.ds(cnt, L)], dlv, mask=m)
            cnt_sm[0] = cnt + jnp.sum(m.astype(jnp.int32))

        cnt = cnt_sm[0]
        zs = jnp.zeros((L,), jnp.int32)
        for i in range(PAIR // L):
            osrc_v[pl.ds(cnt + i * L, L)] = zs + ZROW
            odst_v[pl.ds(cnt + i * L, L)] = zs
        npairs = lax.shift_right_logical(cnt + PAIR - 1, 8)
        cnts_v[pl.ds(g * L, L)] = zs + npairs

        @pl.loop(0, npairs)
        def _(j):
            pltpu.async_copy(osrc_v.at[pl.ds(j * PAIR, PAIR)],
                             bsrc_hbm.at[g, w, pl.ds(j * PAIR, PAIR)],
                             sem).wait()
            pltpu.async_copy(odst_v.at[pl.ds(j * PAIR, PAIR)],
                             bdst_hbm.at[g, w, pl.ds(j * PAIR, PAIR)],
                             sem).wait()

    pltpu.async_copy(cnts_v, bcnt_hbm.at[w], sem).wait()


@jax.jit
def _bucket(edges_flat):
    k = pl.kernel(
        _bucket_body,
        out_type=(
            jax.ShapeDtypeStruct((NB, NC * NS, CAPW), jnp.int32),
            jax.ShapeDtypeStruct((NB, NC * NS, CAPW), jnp.int32),
            jax.ShapeDtypeStruct((NC * NS, NB * L), jnp.int32),
        ),
        mesh=_VMESH,
        compiler_params=_SC_PARAMS,
        scratch_types=[
            pltpu.VMEM((EPW,), jnp.int32),
            pltpu.VMEM((EPW,), jnp.int32),
            pltpu.VMEM((CAPW,), jnp.int32),
            pltpu.VMEM((CAPW,), jnp.int32),
            pltpu.VMEM((NB * L,), jnp.int32),
            pltpu.SMEM((1,), jnp.int32),
            pltpu.SemaphoreType.DMA,
        ],
    )
    return k(edges_flat)


def _agg_body(xs_hbm, bsrc, bdst, bcnt, zeros_hbm, agg_hbm,
              xs_sp, agg_sp, sidx0, sidx1, didx0, didx1, buf0, buf1,
              cnt_v0, cnt_v1, sem_i, sem_g, sem_s0, sem_s1):
    c = lax.axis_index("c")
    s = lax.axis_index("s")
    xrows = NPADX // NS
    arows = RNG // NS

    pltpu.async_copy(xs_hbm.at[pl.ds(s * xrows, xrows)],
                     xs_sp.at[pl.ds(s * xrows, xrows)], sem_i).wait()
    pltpu.async_copy(bcnt.at[2 * s], cnt_v0, sem_i).wait()
    pltpu.async_copy(bcnt.at[2 * s + 1], cnt_v1, sem_i).wait()

    for p in range(PHASES):
        g = c * PHASES + p
        pltpu.async_copy(zeros_hbm.at[pl.ds(s * arows, arows)],
                         agg_sp.at[pl.ds(s * arows, arows)], sem_i).wait()
        plsc.subcore_barrier()

        for wi in range(2):
            w = 2 * s + wi
            cnt_v = cnt_v0 if wi == 0 else cnt_v1
            npairs = jnp.max(cnt_v[pl.ds(g * L, L)])

            @pl.loop(0, npairs)
            def _(j):
                @pl.when(j > 0)
                def _():
                    pltpu.make_async_copy(buf0, agg_sp.at[didx0],
                                          sem_s0).wait()
                    pltpu.make_async_copy(buf1, agg_sp.at[didx1],
                                          sem_s1).wait()
                base = j * PAIR
                pltpu.async_copy(bsrc.at[g, w, pl.ds(base, CHUNK)], sidx0,
                                 sem_i).wait()
                pltpu.async_copy(bdst.at[g, w, pl.ds(base, CHUNK)], didx0,
                                 sem_i).wait()
                pltpu.async_copy(bsrc.at[g, w, pl.ds(base + CHUNK, CHUNK)],
                                 sidx1, sem_i).wait()
                pltpu.async_copy(bdst.at[g, w, pl.ds(base + CHUNK, CHUNK)],
                                 didx1, sem_i).wait()
                pltpu.async_copy(xs_sp.at[sidx0], buf0, sem_g).wait()
                pltpu.async_copy(buf0, agg_sp.at[didx0], sem_s0, add=True)
                pltpu.async_copy(xs_sp.at[sidx1], buf1, sem_g).wait()
                pltpu.async_copy(buf1, agg_sp.at[didx1], sem_s1, add=True)

            @pl.when(npairs > 0)
            def _():
                pltpu.make_async_copy(buf0, agg_sp.at[didx0], sem_s0).wait()
                pltpu.make_async_copy(buf1, agg_sp.at[didx1], sem_s1).wait()

        plsc.subcore_barrier()
        pltpu.async_copy(agg_sp.at[pl.ds(s * arows, arows)],
                         agg_hbm.at[pl.ds(g * RNG + s * arows, arows)],
                         sem_i).wait()


@jax.jit
def _aggregate(xs_pad, bsrc, bdst, bcnt, zeros_rows):
    k = pl.kernel(
        _agg_body,
        out_type=jax.ShapeDtypeStruct((NPADX, F), jnp.float32),
        mesh=_VMESH,
        compiler_params=_SC_PARAMS,
        scratch_types=[
            pltpu.VMEM_SHARED((NPADX, F), jnp.float32),
            pltpu.VMEM_SHARED((RNG, F), jnp.float32),
            pltpu.VMEM((CHUNK,), jnp.int32),
            pltpu.VMEM((CHUNK,), jnp.int32),
            pltpu.VMEM((CHUNK,), jnp.int32),
            pltpu.VMEM((CHUNK,), jnp.int32),
            pltpu.VMEM((CHUNK, F), jnp.float32),
            pltpu.VMEM((CHUNK, F), jnp.float32),
            pltpu.VMEM((NB * L,), jnp.int32),
            pltpu.VMEM((NB * L,), jnp.int32),
            pltpu.SemaphoreType.DMA,
            pltpu.SemaphoreType.DMA,
            pltpu.SemaphoreType.DMA,
            pltpu.SemaphoreType.DMA,
        ],
    )
    return k(xs_pad, bsrc, bdst, bcnt, zeros_rows)


def _final_body(agg_ref, inorm_ref, onorm_ref, wps_ref, w1_ref, b1_ref,
                w2_ref, b2_ref, out_ref):
    a = agg_ref[:N, :] * inorm_ref[...]
    z = jnp.dot(a, w1_ref[...], preferred_element_type=jnp.float32,
                precision=lax.Precision.HIGHEST) + b1_ref[...]
    h = jnp.maximum(z, 0.0)
    wpre = jnp.sum(wps_ref[...], axis=0, keepdims=True)[:, :N]
    w_row = wpre * onorm_ref[...]
    v = jnp.dot(w_row, h, preferred_element_type=jnp.float32,
                precision=lax.Precision.HIGHEST) * (1.0 / N)
    out_ref[...] = jnp.dot(v, w2_ref[...], preferred_element_type=jnp.float32,
                           precision=lax.Precision.HIGHEST) + b2_ref[...]


@jax.jit
def _final(agg, inorm_col, onorm_row, wps2, W1, b1, W2, b2):
    return pl.pallas_call(
        _final_body,
        out_shape=jax.ShapeDtypeStruct((1, C), jnp.float32),
    )(agg, inorm_col, onorm_row, wps2, W1, b1, W2, b2)


def kernel(in_feat, edge_index, W1, b1, W2, b2):
    edges_flat = edge_index.reshape(2 * E)

    degp = _degrees(edge_index)
    norms = _norms(degp)

    onorm_col = norms[0].reshape(NHIST, 1)[:N]
    inorm_col = norms[1].reshape(NHIST, 1)[:N]
    onorm_row = norms[0:1, :N]
    inorm_flat = norms[1]

    xs = _scale(in_feat, onorm_col)
    wps = _w_partials(edges_flat, inorm_flat)
    bsrc, bdst, bcnt = _bucket(edges_flat)

    xs_pad = jnp.concatenate(
        [xs, jnp.zeros((NPADX - N, F), jnp.float32)])
    zeros_rows = jnp.zeros((RNG, F), jnp.float32)
    agg = _aggregate(xs_pad, bsrc, bdst, bcnt, zeros_rows)

    wps2 = wps.reshape(NC * NS, NHIST)
    out = _final(agg, inorm_col, onorm_row, wps2, W1,
                 b1.reshape(1, H), W2, b2.reshape(1, C))
    return out


# speedup vs baseline: 11.2753x; 1.0941x over previous
import dataclasses
import functools

import jax
import jax.numpy as jnp
from jax import lax
from jax.experimental import pallas as pl
from jax.experimental.pallas import tpu as pltpu
from jax.experimental.pallas import tpu_sc as plsc

N = 10000
E = 320000
F = 128
H = 256
C = 64

NC = 2
NS = 16
L = 16

NHIST = 10240
NPADX = 10240
NB = 8
RNG = NPADX // NB
PHASES = NB // NC
EPW = E // (NC * NS)
CHUNK = 128
PAIR = 2 * CHUNK
CAPW = 10496
ZROW = N

_VMESH = plsc.VectorSubcoreMesh(core_axis_name="c", subcore_axis_name="s")

_SC_PARAMS = pltpu.CompilerParams()
if "needs_layout_passes" in pltpu.CompilerParams.__dataclass_fields__:
    _SC_PARAMS = dataclasses.replace(_SC_PARAMS, needs_layout_passes=False)


def _degree_body(edges_hbm, out_hbm, idx_v, hist_v, sem):
    c = lax.axis_index("c")
    s = lax.axis_index("s")
    epw = E // NS

    @pl.loop(0, NHIST, step=L)
    def _(i):
        hist_v[pl.ds(i, L)] = jnp.zeros((L,), jnp.float32)

    pltpu.async_copy(edges_hbm.at[pl.ds(c * E + s * epw, epw)], idx_v,
                     sem).wait()
    ones = jnp.full((L,), 1.0, jnp.float32)

    @pl.loop(0, epw, step=L)
    def _(i):
        iv = idx_v[pl.ds(i, L)]
        plsc.addupdate_scatter(hist_v, [iv], ones)

    pltpu.async_copy(hist_v, out_hbm.at[c, s], sem).wait()


@jax.jit
def _degrees(edge_index):
    k = pl.kernel(
        _degree_body,
        out_type=jax.ShapeDtypeStruct((NC, NS, NHIST), jnp.float32),
        mesh=_VMESH,
        compiler_params=_SC_PARAMS,
        scratch_types=[
            pltpu.VMEM((E // NS,), jnp.int32),
            pltpu.VMEM((NHIST,), jnp.float32),
            pltpu.SemaphoreType.DMA,
        ],
    )
    return k(edge_index.reshape(2 * E))


def _norms_body(degp_ref, norms_ref):
    deg = jnp.sum(degp_ref[...], axis=1)
    lane = lax.broadcasted_iota(jnp.int32, (NC, NHIST), 1)
    norm = lax.rsqrt(jnp.maximum(deg, 1.0))
    norms_ref[...] = jnp.where(lane < N, norm, 0.0)


@jax.jit
def _norms(degp):
    return pl.pallas_call(
        _norms_body,
        out_shape=jax.ShapeDtypeStruct((NC, NHIST), jnp.float32),
    )(degp)


def _scale_body(x_ref, onorm_ref, xs_ref):
    xs_ref[...] = x_ref[...] * onorm_ref[...]


@jax.jit
def _scale(in_feat, onorm_col):
    return pl.pallas_call(
        _scale_body,
        out_shape=jax.ShapeDtypeStruct((N, F), jnp.float32),
    )(in_feat, onorm_col)


def _w_body(edges_hbm, inorm_hbm, wps_hbm, src_v, dst_v, inorm_v, whist_v,
            sem):
    c = lax.axis_index("c")
    s = lax.axis_index("s")
    w = c * NS + s

    @pl.loop(0, NHIST, step=L)
    def _(i):
        whist_v[pl.ds(i, L)] = jnp.zeros((L,), jnp.float32)

    pltpu.async_copy(edges_hbm.at[pl.ds(w * EPW, EPW)], src_v, sem).wait()
    pltpu.async_copy(edges_hbm.at[pl.ds(E + w * EPW, EPW)], dst_v,
                     sem).wait()
    pltpu.async_copy(inorm_hbm, inorm_v, sem).wait()

    @pl.loop(0, EPW, step=L)
    def _(i):
        dv = dst_v[pl.ds(i, L)]
        sv = src_v[pl.ds(i, L)]
        vals = plsc.load_gather(inorm_v, [dv])
        plsc.addupdate_scatter(whist_v, [sv], vals)

    pltpu.async_copy(whist_v, wps_hbm.at[c, s], sem).wait()


@jax.jit
def _w_partials(edges_flat, inorm_flat):
    k = pl.kernel(
        _w_body,
        out_type=jax.ShapeDtypeStruct((NC, NS, NHIST), jnp.float32),
        mesh=_VMESH,
        compiler_params=_SC_PARAMS,
        scratch_types=[
            pltpu.VMEM((EPW,), jnp.int32),
            pltpu.VMEM((EPW,), jnp.int32),
            pltpu.VMEM((NHIST,), jnp.float32),
            pltpu.VMEM((NHIST,), jnp.float32),
            pltpu.SemaphoreType.DMA,
        ],
    )
    return k(edges_flat, inorm_flat)


def _bucket_body(edges_hbm, bsrc_hbm, bdst_hbm, bcnt_hbm,
                 src_v, dst_v, os0, os1, os2, os3, od0, od1, od2, od3,
                 cnts_v, cnt_sm, sem):
    c = lax.axis_index("c")
    s = lax.axis_index("s")
    w = c * NS + s
    osrc = [os0, os1, os2, os3]
    odst = [od0, od1, od2, od3]

    pltpu.async_copy(edges_hbm.at[pl.ds(w * EPW, EPW)], src_v, sem).wait()
    pltpu.async_copy(edges_hbm.at[pl.ds(E + w * EPW, EPW)], dst_v,
                     sem).wait()

    for half in range(2):
        for q in range(4):
            cnt_sm[q] = 0

        @pl.loop(0, EPW, step=L)
        def _(i):
            dv = dst_v[pl.ds(i, L)]
            sv = src_v[pl.ds(i, L)]
            bv = lax.shift_right_logical(dv * 13108, 24)
            for q in range(4):
                g = 4 * half + q
                m = bv == g
                cnt = cnt_sm[q]
                plsc.store_compressed(osrc[q].at[pl.ds(cnt, L)], sv, mask=m)
                plsc.store_compressed(odst[q].at[pl.ds(cnt, L)],
                                      dv - g * RNG, mask=m)
                cnt_sm[q] = cnt + jnp.sum(m.astype(jnp.int32))

        zs = jnp.zeros((L,), jnp.int32)
        for q in range(4):
            g = 4 * half + q
            cnt = cnt_sm[q]
            for i in range(PAIR // L):
                osrc[q][pl.ds(cnt + i * L, L)] = zs + ZROW
                odst[q][pl.ds(cnt + i * L, L)] = zs
            npairs = lax.shift_right_logical(cnt + PAIR - 1, 8)
            cnts_v[pl.ds(g * L, L)] = zs + npairs

            @pl.loop(0, npairs)
            def _(j):
                pltpu.async_copy(osrc[q].at[pl.ds(j * PAIR, PAIR)],
                                 bsrc_hbm.at[g, w, pl.ds(j * PAIR, PAIR)],
                                 sem).wait()
                pltpu.async_copy(odst[q].at[pl.ds(j * PAIR, PAIR)],
                                 bdst_hbm.at[g, w, pl.ds(j * PAIR, PAIR)],
                                 sem).wait()

    pltpu.async_copy(cnts_v, bcnt_hbm.at[w], sem).wait()


@jax.jit
def _bucket(edges_flat):
    k = pl.kernel(
        _bucket_body,
        out_type=(
            jax.ShapeDtypeStruct((NB, NC * NS, CAPW), jnp.int32),
            jax.ShapeDtypeStruct((NB, NC * NS, CAPW), jnp.int32),
            jax.ShapeDtypeStruct((NC * NS, NB * L), jnp.int32),
        ),
        mesh=_VMESH,
        compiler_params=_SC_PARAMS,
        scratch_types=(
            [pltpu.VMEM((EPW,), jnp.int32)] * 2
            + [pltpu.VMEM((CAPW,), jnp.int32)] * 8
            + [pltpu.VMEM((NB * L,), jnp.int32),
               pltpu.SMEM((4,), jnp.int32),
               pltpu.SemaphoreType.DMA]
        ),
    )
    return k(edges_flat)


def _agg_body(xs_hbm, bsrc, bdst, bcnt, zeros_hbm, agg_hbm,
              xs_sp, agg_sp, sidx0, sidx1, didx0, didx1, buf0, buf1,
              cnt_v0, cnt_v1, sem_i, sem_g, sem_s0, sem_s1):
    c = lax.axis_index("c")
    s = lax.axis_index("s")
    xrows = NPADX // NS
    arows = RNG // NS

    pltpu.async_copy(xs_hbm.at[pl.ds(s * xrows, xrows)],
                     xs_sp.at[pl.ds(s * xrows, xrows)], sem_i).wait()
    pltpu.async_copy(bcnt.at[2 * s], cnt_v0, sem_i).wait()
    pltpu.async_copy(bcnt.at[2 * s + 1], cnt_v1, sem_i).wait()

    for p in range(PHASES):
        g = c * PHASES + p
        pltpu.async_copy(zeros_hbm.at[pl.ds(s * arows, arows)],
                         agg_sp.at[pl.ds(s * arows, arows)], sem_i).wait()
        plsc.subcore_barrier()

        for wi in range(2):
            w = 2 * s + wi
            cnt_v = cnt_v0 if wi == 0 else cnt_v1
            npairs = jnp.max(cnt_v[pl.ds(g * L, L)])

            @pl.loop(0, npairs)
            def _(j):
                @pl.when(j > 0)
                def _():
                    pltpu.make_async_copy(buf0, agg_sp.at[didx0],
                                          sem_s0).wait()
                    pltpu.make_async_copy(buf1, agg_sp.at[didx1],
                                          sem_s1).wait()
                base = j * PAIR
                pltpu.async_copy(bsrc.at[g, w, pl.ds(base, CHUNK)], sidx0,
                                 sem_i).wait()
                pltpu.async_copy(bdst.at[g, w, pl.ds(base, CHUNK)], didx0,
                                 sem_i).wait()
                pltpu.async_copy(bsrc.at[g, w, pl.ds(base + CHUNK, CHUNK)],
                                 sidx1, sem_i).wait()
                pltpu.async_copy(bdst.at[g, w, pl.ds(base + CHUNK, CHUNK)],
                                 didx1, sem_i).wait()
                pltpu.async_copy(xs_sp.at[sidx0], buf0, sem_g).wait()
                pltpu.async_copy(buf0, agg_sp.at[didx0], sem_s0, add=True)
                pltpu.async_copy(xs_sp.at[sidx1], buf1, sem_g).wait()
                pltpu.async_copy(buf1, agg_sp.at[didx1], sem_s1, add=True)

            @pl.when(npairs > 0)
            def _():
                pltpu.make_async_copy(buf0, agg_sp.at[didx0], sem_s0).wait()
                pltpu.make_async_copy(buf1, agg_sp.at[didx1], sem_s1).wait()

        plsc.subcore_barrier()
        pltpu.async_copy(agg_sp.at[pl.ds(s * arows, arows)],
                         agg_hbm.at[pl.ds(g * RNG + s * arows, arows)],
                         sem_i).wait()


@jax.jit
def _aggregate(xs_pad, bsrc, bdst, bcnt, zeros_rows):
    k = pl.kernel(
        _agg_body,
        out_type=jax.ShapeDtypeStruct((NPADX, F), jnp.float32),
        mesh=_VMESH,
        compiler_params=_SC_PARAMS,
        scratch_types=[
            pltpu.VMEM_SHARED((NPADX, F), jnp.float32),
            pltpu.VMEM_SHARED((RNG, F), jnp.float32),
            pltpu.VMEM((CHUNK,), jnp.int32),
            pltpu.VMEM((CHUNK,), jnp.int32),
            pltpu.VMEM((CHUNK,), jnp.int32),
            pltpu.VMEM((CHUNK,), jnp.int32),
            pltpu.VMEM((CHUNK, F), jnp.float32),
            pltpu.VMEM((CHUNK, F), jnp.float32),
            pltpu.VMEM((NB * L,), jnp.int32),
            pltpu.VMEM((NB * L,), jnp.int32),
            pltpu.SemaphoreType.DMA,
            pltpu.SemaphoreType.DMA,
            pltpu.SemaphoreType.DMA,
            pltpu.SemaphoreType.DMA,
        ],
    )
    return k(xs_pad, bsrc, bdst, bcnt, zeros_rows)


def _final_body(agg_ref, inorm_ref, onorm_ref, wps_ref, w1_ref, b1_ref,
                w2_ref, b2_ref, out_ref):
    a = agg_ref[:N, :] * inorm_ref[...]
    z = jnp.dot(a, w1_ref[...], preferred_element_type=jnp.float32,
                precision=lax.Precision.HIGHEST) + b1_ref[...]
    h = jnp.maximum(z, 0.0)
    wpre = jnp.sum(wps_ref[...], axis=0, keepdims=True)[:, :N]
    w_row = wpre * onorm_ref[...]
    v = jnp.dot(w_row, h, preferred_element_type=jnp.float32,
                precision=lax.Precision.HIGHEST) * (1.0 / N)
    out_ref[...] = jnp.dot(v, w2_ref[...], preferred_element_type=jnp.float32,
                           precision=lax.Precision.HIGHEST) + b2_ref[...]


@jax.jit
def _final(agg, inorm_col, onorm_row, wps2, W1, b1, W2, b2):
    return pl.pallas_call(
        _final_body,
        out_shape=jax.ShapeDtypeStruct((1, C), jnp.float32),
    )(agg, inorm_col, onorm_row, wps2, W1, b1, W2, b2)


def kernel(in_feat, edge_index, W1, b1, W2, b2):
    edges_flat = edge_index.reshape(2 * E)

    degp = _degrees(edge_index)
    norms = _norms(degp)

    onorm_col = norms[0].reshape(NHIST, 1)[:N]
    inorm_col = norms[1].reshape(NHIST, 1)[:N]
    onorm_row = norms[0:1, :N]
    inorm_flat = norms[1]

    xs = _scale(in_feat, onorm_col)
    wps = _w_partials(edges_flat, inorm_flat)
    bsrc, bdst, bcnt = _bucket(edges_flat)

    xs_pad = jnp.concatenate(
        [xs, jnp.zeros((NPADX - N, F), jnp.float32)])
    zeros_rows = jnp.zeros((RNG, F), jnp.float32)
    agg = _aggregate(xs_pad, bsrc, bdst, bcnt, zeros_rows)

    wps2 = wps.reshape(NC * NS, NHIST)
    out = _final(agg, inorm_col, onorm_row, wps2, W1,
                 b1.reshape(1, H), W2, b2.reshape(1, C))
    return out


# speedup vs baseline: 13.8024x; 1.2241x over previous
import dataclasses
import functools

import jax
import jax.numpy as jnp
from jax import lax
from jax.experimental import pallas as pl
from jax.experimental.pallas import tpu as pltpu
from jax.experimental.pallas import tpu_sc as plsc

N = 10000
E = 320000
F = 128
H = 256
C = 64

NC = 2
NS = 16
L = 16

NHIST = 10240
NPADX = 10240
NB = 8
RNG = NPADX // NB
PHASES = NB // NC
EPW = E // (NC * NS)
CHUNK = 128
PAIR = 2 * CHUNK
CAPW = 10496
ZROW = N

_VMESH = plsc.VectorSubcoreMesh(core_axis_name="c", subcore_axis_name="s")

_SC_PARAMS = pltpu.CompilerParams()
if "needs_layout_passes" in pltpu.CompilerParams.__dataclass_fields__:
    _SC_PARAMS = dataclasses.replace(_SC_PARAMS, needs_layout_passes=False)


def _degree_body(edges_hbm, out_hbm, idx_v, hist_v, sem):
    c = lax.axis_index("c")
    s = lax.axis_index("s")
    epw = E // NS

    @pl.loop(0, NHIST, step=L)
    def _(i):
        hist_v[pl.ds(i, L)] = jnp.zeros((L,), jnp.float32)

    pltpu.async_copy(edges_hbm.at[pl.ds(c * E + s * epw, epw)], idx_v,
                     sem).wait()
    ones = jnp.full((L,), 1.0, jnp.float32)

    @pl.loop(0, epw, step=L)
    def _(i):
        iv = idx_v[pl.ds(i, L)]
        plsc.addupdate_scatter(hist_v, [iv], ones)

    pltpu.async_copy(hist_v, out_hbm.at[c, s], sem).wait()


@jax.jit
def _degrees(edge_index):
    k = pl.kernel(
        _degree_body,
        out_type=jax.ShapeDtypeStruct((NC, NS, NHIST), jnp.float32),
        mesh=_VMESH,
        compiler_params=_SC_PARAMS,
        scratch_types=[
            pltpu.VMEM((E // NS,), jnp.int32),
            pltpu.VMEM((NHIST,), jnp.float32),
            pltpu.SemaphoreType.DMA,
        ],
    )
    return k(edge_index.reshape(2 * E))


def _norms_body(degp_ref, norms_ref):
    deg = jnp.sum(degp_ref[...], axis=1)
    lane = lax.broadcasted_iota(jnp.int32, (NC, NHIST), 1)
    norm = lax.rsqrt(jnp.maximum(deg, 1.0))
    norms_ref[...] = jnp.where(lane < N, norm, 0.0)


@jax.jit
def _norms(degp):
    return pl.pallas_call(
        _norms_body,
        out_shape=jax.ShapeDtypeStruct((NC, NHIST), jnp.float32),
    )(degp)


def _scale_body(x_ref, onorm_ref, xs_ref):
    xs_ref[...] = x_ref[...] * onorm_ref[...]


@jax.jit
def _scale(in_feat, onorm_col):
    return pl.pallas_call(
        _scale_body,
        out_shape=jax.ShapeDtypeStruct((N, F), jnp.float32),
    )(in_feat, onorm_col)


def _w_body(edges_hbm, inorm_hbm, wps_hbm, src_v, dst_v, inorm_v, whist_v,
            sem):
    c = lax.axis_index("c")
    s = lax.axis_index("s")
    w = c * NS + s

    @pl.loop(0, NHIST, step=L)
    def _(i):
        whist_v[pl.ds(i, L)] = jnp.zeros((L,), jnp.float32)

    pltpu.async_copy(edges_hbm.at[pl.ds(w * EPW, EPW)], src_v, sem).wait()
    pltpu.async_copy(edges_hbm.at[pl.ds(E + w * EPW, EPW)], dst_v,
                     sem).wait()
    pltpu.async_copy(inorm_hbm, inorm_v, sem).wait()

    @pl.loop(0, EPW, step=L)
    def _(i):
        dv = dst_v[pl.ds(i, L)]
        sv = src_v[pl.ds(i, L)]
        vals = plsc.load_gather(inorm_v, [dv])
        plsc.addupdate_scatter(whist_v, [sv], vals)

    pltpu.async_copy(whist_v, wps_hbm.at[c, s], sem).wait()


@jax.jit
def _w_partials(edges_flat, inorm_flat):
    k = pl.kernel(
        _w_body,
        out_type=jax.ShapeDtypeStruct((NC, NS, NHIST), jnp.float32),
        mesh=_VMESH,
        compiler_params=_SC_PARAMS,
        scratch_types=[
            pltpu.VMEM((EPW,), jnp.int32),
            pltpu.VMEM((EPW,), jnp.int32),
            pltpu.VMEM((NHIST,), jnp.float32),
            pltpu.VMEM((NHIST,), jnp.float32),
            pltpu.SemaphoreType.DMA,
        ],
    )
    return k(edges_flat, inorm_flat)


def _bucket_body(edges_hbm, bsrc_hbm, bdst_hbm, bcnt_hbm,
                 src_v, dst_v, os0, os1, os2, os3, od0, od1, od2, od3,
                 cnts_v, cnt_sm, sem):
    c = lax.axis_index("c")
    s = lax.axis_index("s")
    w = c * NS + s
    osrc = [os0, os1, os2, os3]
    odst = [od0, od1, od2, od3]

    pltpu.async_copy(edges_hbm.at[pl.ds(w * EPW, EPW)], src_v, sem).wait()
    pltpu.async_copy(edges_hbm.at[pl.ds(E + w * EPW, EPW)], dst_v,
                     sem).wait()

    for half in range(2):
        for q in range(4):
            cnt_sm[q] = 0

        @pl.loop(0, EPW, step=L)
        def _(i):
            dv = dst_v[pl.ds(i, L)]
            sv = src_v[pl.ds(i, L)]
            bv = lax.shift_right_logical(dv * 13108, 24)
            for q in range(4):
                g = 4 * half + q
                m = bv == g
                cnt = cnt_sm[q]
                plsc.store_compressed(osrc[q].at[pl.ds(cnt, L)], sv, mask=m)
                plsc.store_compressed(odst[q].at[pl.ds(cnt, L)],
                                      dv - g * RNG, mask=m)
                cnt_sm[q] = cnt + jnp.sum(m.astype(jnp.int32))

        zs = jnp.zeros((L,), jnp.int32)
        for q in range(4):
            g = 4 * half + q
            cnt = cnt_sm[q]
            for i in range(PAIR // L):
                osrc[q][pl.ds(cnt + i * L, L)] = zs + ZROW
                odst[q][pl.ds(cnt + i * L, L)] = zs
            npairs = lax.shift_right_logical(cnt + PAIR - 1, 8)
            cnts_v[pl.ds(g * L, L)] = zs + npairs

            @pl.loop(0, npairs)
            def _(j):
                pltpu.async_copy(osrc[q].at[pl.ds(j * PAIR, PAIR)],
                                 bsrc_hbm.at[g, w, pl.ds(j * PAIR, PAIR)],
                                 sem).wait()
                pltpu.async_copy(odst[q].at[pl.ds(j * PAIR, PAIR)],
                                 bdst_hbm.at[g, w, pl.ds(j * PAIR, PAIR)],
                                 sem).wait()

    pltpu.async_copy(cnts_v, bcnt_hbm.at[w], sem).wait()


@jax.jit
def _bucket(edges_flat):
    k = pl.kernel(
        _bucket_body,
        out_type=(
            jax.ShapeDtypeStruct((NB, NC * NS, CAPW), jnp.int32),
            jax.ShapeDtypeStruct((NB, NC * NS, CAPW), jnp.int32),
            jax.ShapeDtypeStruct((NC * NS, NB * L), jnp.int32),
        ),
        mesh=_VMESH,
        compiler_params=_SC_PARAMS,
        scratch_types=(
            [pltpu.VMEM((EPW,), jnp.int32)] * 2
            + [pltpu.VMEM((CAPW,), jnp.int32)] * 8
            + [pltpu.VMEM((NB * L,), jnp.int32),
               pltpu.SMEM((4,), jnp.int32),
               pltpu.SemaphoreType.DMA]
        ),
    )
    return k(edges_flat)


def _agg_body(xs_hbm, bsrc, bdst, bcnt, zeros_hbm, agg_hbm,
              xs_sp, agg_sp, sidx0, sidx1, didx0, didx1, buf0, buf1,
              cnt_v0, cnt_v1, sem_i, sem_g, sem_s0, sem_s1):
    c = lax.axis_index("c")
    s = lax.axis_index("s")
    xrows = NPADX // NS
    arows = RNG // NS

    pltpu.async_copy(xs_hbm.at[pl.ds(s * xrows, xrows)],
                     xs_sp.at[pl.ds(s * xrows, xrows)], sem_i).wait()
    pltpu.async_copy(bcnt.at[2 * s], cnt_v0, sem_i).wait()
    pltpu.async_copy(bcnt.at[2 * s + 1], cnt_v1, sem_i).wait()

    for p in range(PHASES):
        g = c * PHASES + p
        pltpu.async_copy(zeros_hbm.at[pl.ds(s * arows, arows)],
                         agg_sp.at[pl.ds(s * arows, arows)], sem_i).wait()
        plsc.subcore_barrier()

        for wi in range(2):
            w = 2 * s + wi
            cnt_v = cnt_v0 if wi == 0 else cnt_v1
            npairs = jnp.max(cnt_v[pl.ds(g * L, L)])

            @pl.loop(0, npairs)
            def _(j):
                @pl.when(j > 0)
                def _():
                    pltpu.make_async_copy(buf0, agg_sp.at[didx0],
                                          sem_s0).wait()
                    pltpu.make_async_copy(buf1, agg_sp.at[didx1],
                                          sem_s1).wait()
                base = j * PAIR
                pltpu.async_copy(bsrc.at[g, w, pl.ds(base, CHUNK)], sidx0,
                                 sem_i)
                pltpu.async_copy(bdst.at[g, w, pl.ds(base, CHUNK)], didx0,
                                 sem_i)
                pltpu.async_copy(bsrc.at[g, w, pl.ds(base + CHUNK, CHUNK)],
                                 sidx1, sem_i)
                pltpu.async_copy(bdst.at[g, w, pl.ds(base + CHUNK, CHUNK)],
                                 didx1, sem_i)
                pltpu.make_async_copy(bsrc.at[g, w, pl.ds(base, CHUNK)],
                                      sidx0, sem_i).wait()
                pltpu.make_async_copy(bdst.at[g, w, pl.ds(base, CHUNK)],
                                      didx0, sem_i).wait()
                pltpu.make_async_copy(bsrc.at[g, w,
                                              pl.ds(base + CHUNK, CHUNK)],
                                      sidx1, sem_i).wait()
                pltpu.make_async_copy(bdst.at[g, w,
                                              pl.ds(base + CHUNK, CHUNK)],
                                      didx1, sem_i).wait()
                pltpu.async_copy(xs_sp.at[sidx0], buf0, sem_g).wait()
                pltpu.async_copy(buf0, agg_sp.at[didx0], sem_s0, add=True)
                pltpu.async_copy(xs_sp.at[sidx1], buf1, sem_g).wait()
                pltpu.async_copy(buf1, agg_sp.at[didx1], sem_s1, add=True)

            @pl.when(npairs > 0)
            def _():
                pltpu.make_async_copy(buf0, agg_sp.at[didx0], sem_s0).wait()
                pltpu.make_async_copy(buf1, agg_sp.at[didx1], sem_s1).wait()

        plsc.subcore_barrier()
        pltpu.async_copy(agg_sp.at[pl.ds(s * arows, arows)],
                         agg_hbm.at[pl.ds(g * RNG + s * arows, arows)],
                         sem_i).wait()


@jax.jit
def _aggregate(xs_pad, bsrc, bdst, bcnt, zeros_rows):
    k = pl.kernel(
        _agg_body,
        out_type=jax.ShapeDtypeStruct((NPADX, F), jnp.float32),
        mesh=_VMESH,
        compiler_params=_SC_PARAMS,
        scratch_types=[
            pltpu.VMEM_SHARED((NPADX, F), jnp.float32),
            pltpu.VMEM_SHARED((RNG, F), jnp.float32),
            pltpu.VMEM((CHUNK,), jnp.int32),
            pltpu.VMEM((CHUNK,), jnp.int32),
            pltpu.VMEM((CHUNK,), jnp.int32),
            pltpu.VMEM((CHUNK,), jnp.int32),
            pltpu.VMEM((CHUNK, F), jnp.float32),
            pltpu.VMEM((CHUNK, F), jnp.float32),
            pltpu.VMEM((NB * L,), jnp.int32),
            pltpu.VMEM((NB * L,), jnp.int32),
            pltpu.SemaphoreType.DMA,
            pltpu.SemaphoreType.DMA,
            pltpu.SemaphoreType.DMA,
            pltpu.SemaphoreType.DMA,
        ],
    )
    return k(xs_pad, bsrc, bdst, bcnt, zeros_rows)


def _final_body(agg_ref, inorm_ref, onorm_ref, wps_ref, w1_ref, b1_ref,
                w2_ref, b2_ref, out_ref):
    a = agg_ref[:N, :] * inorm_ref[...]
    z = jnp.dot(a, w1_ref[...], preferred_element_type=jnp.float32,
                precision=lax.Precision.HIGHEST) + b1_ref[...]
    h = jnp.maximum(z, 0.0)
    wpre = jnp.sum(wps_ref[...], axis=0, keepdims=True)[:, :N]
    w_row = wpre * onorm_ref[...]
    v = jnp.dot(w_row, h, preferred_element_type=jnp.float32,
                precision=lax.Precision.HIGHEST) * (1.0 / N)
    out_ref[...] = jnp.dot(v, w2_ref[...], preferred_element_type=jnp.float32,
                           precision=lax.Precision.HIGHEST) + b2_ref[...]


@jax.jit
def _final(agg, inorm_col, onorm_row, wps2, W1, b1, W2, b2):
    return pl.pallas_call(
        _final_body,
        out_shape=jax.ShapeDtypeStruct((1, C), jnp.float32),
    )(agg, inorm_col, onorm_row, wps2, W1, b1, W2, b2)


def kernel(in_feat, edge_index, W1, b1, W2, b2):
    edges_flat = edge_index.reshape(2 * E)

    degp = _degrees(edge_index)
    norms = _norms(degp)

    onorm_col = norms[0].reshape(NHIST, 1)[:N]
    inorm_col = norms[1].reshape(NHIST, 1)[:N]
    onorm_row = norms[0:1, :N]
    inorm_flat = norms[1]

    xs = _scale(in_feat, onorm_col)
    wps = _w_partials(edges_flat, inorm_flat)
    bsrc, bdst, bcnt = _bucket(edges_flat)

    xs_pad = jnp.concatenate(
        [xs, jnp.zeros((NPADX - N, F), jnp.float32)])
    zeros_rows = jnp.zeros((RNG, F), jnp.float32)
    agg = _aggregate(xs_pad, bsrc, bdst, bcnt, zeros_rows)

    wps2 = wps.reshape(NC * NS, NHIST)
    out = _final(agg, inorm_col, onorm_row, wps2, W1,
                 b1.reshape(1, H), W2, b2.reshape(1, C))
    return out
